# Initial kernel scaffold; baseline (speedup 1.0000x reference)
import jax
import jax.numpy as jnp
from jax.experimental import pallas as pl


def kernel(x, base_w1, spline_w1, scaler1, bias1, base_w2, spline_w2, scaler2, bias2, base_w3, spline_w3, scaler3, edge_index, batch):
    raise NotImplementedError("write your pallas kernel here")


# speedup vs baseline: 11.4565x; 11.4565x over previous
import functools

import jax
import jax.numpy as jnp
from jax import lax
from jax.experimental import pallas as pl
from jax.experimental.pallas import tpu as pltpu
from jax.experimental.pallas import tpu_sc as plsc

N = 10000
E = 320000
D = 128
C = 10
NG = 64
NB = 7

NC = 2
NS = 16
NW = NC * NS
EW = E // NW
KS = 80
CHS = EW // KS
RPTA = 640
RPTL = N - RPTA * (NS - 1)

BN = 1000
GRID = N // BN


@functools.cache
def _sc_kernels():
    mesh = plsc.VectorSubcoreMesh(core_axis_name="c", subcore_axis_name="s")

    @functools.partial(
        pl.kernel,
        out_type=jax.ShapeDtypeStruct((NC, N, D), jnp.float32),
        mesh=mesh,
        scratch_types=[
            pltpu.VMEM((CHS, KS), jnp.int32),
            pltpu.VMEM((CHS, KS), jnp.int32),
            pltpu.VMEM((KS, D), jnp.float32),
            pltpu.VMEM_SHARED((N, D), jnp.float32),
            pltpu.SemaphoreType.DMA,
        ],
    )
    def sc_scatter(g_hbm, row_hbm, col_hbm, zeros_hbm, out_hbm, rowv, colv,
                   buf, acc, sem):
        c = lax.axis_index("c")
        s = lax.axis_index("s")
        wid = s * NC + c
        @pl.when(s < NS - 1)
        def _():
            pltpu.sync_copy(zeros_hbm, acc.at[pl.ds(s * RPTA, RPTA)])

        @pl.when(s == NS - 1)
        def _():
            pltpu.sync_copy(zeros_hbm.at[pl.ds(0, RPTL)],
                            acc.at[pl.ds((NS - 1) * RPTA, RPTL)])

        pltpu.sync_copy(row_hbm.at[wid], rowv)
        pltpu.sync_copy(col_hbm.at[wid], colv)
        plsc.subcore_barrier()

        def body(j, carry):
            pltpu.async_copy(g_hbm.at[rowv.at[j]], buf, sem).wait()
            pltpu.sync_copy(buf, acc.at[colv.at[j]], add=True)
            return carry

        lax.fori_loop(0, CHS, body, 0)
        plsc.subcore_barrier()

        @pl.when(s < NS - 1)
        def _():
            pltpu.sync_copy(acc.at[pl.ds(s * RPTA, RPTA)],
                            out_hbm.at[c, pl.ds(s * RPTA, RPTA)])

        @pl.when(s == NS - 1)
        def _():
            pltpu.sync_copy(acc.at[pl.ds((NS - 1) * RPTA, RPTL)],
                            out_hbm.at[c, pl.ds((NS - 1) * RPTA, RPTL)])

    return sc_scatter


def _silu(v):
    return v * jax.nn.sigmoid(v)


def _bspline_bases(x):
    t = [0.5 * i - 2.5 for i in range(11)]
    b = [jnp.where((x >= t[i]) & (x < t[i + 1]), 1.0, 0.0) for i in range(10)]
    for k in range(1, 4):
        b = [
            (x - t[i]) / (t[i + k] - t[i]) * b[i]
            + (t[i + k + 1] - x) / (t[i + k + 1] - t[i + 1]) * b[i + 1]
            for i in range(10 - k)
        ]
    return b


def _kan(h, wcat):
    feats = jnp.concatenate([_silu(h)] + _bspline_bases(h), axis=1)
    return jnp.dot(feats, wcat, preferred_element_type=jnp.float32,
                   precision=lax.Precision.HIGHEST)


def _kan1_body(x_ref, d0_ref, d1_ref, wcat_ref, g_ref, dinv_ref):
    dinv = lax.rsqrt(d0_ref[...] + d1_ref[...] + 1.0)
    h = _kan(x_ref[...], wcat_ref[...])
    g_ref[...] = h * dinv
    dinv_ref[...] = dinv


def _mid_body(s0_ref, s1_ref, g_ref, dinv_ref, bias_ref, wcat_ref, out_ref):
    dinv = dinv_ref[...]
    u = (s0_ref[...] + s1_ref[...] + g_ref[...]) * dinv + bias_ref[...]
    out_ref[...] = _kan(_silu(u), wcat_ref[...]) * dinv


def _final_body(s0_ref, s1_ref, g_ref, dinv_ref, bias_ref, batch_ref,
                wcat_ref, out_ref, acc, cnt):
    i = pl.program_id(0)

    @pl.when(i == 0)
    def _():
        acc[...] = jnp.zeros_like(acc)
        cnt[...] = jnp.zeros_like(cnt)

    dinv = dinv_ref[...]
    u = (s0_ref[...] + s1_ref[...] + g_ref[...]) * dinv + bias_ref[...]
    h = _silu(u)
    onehot = jnp.where(
        batch_ref[...] == lax.broadcasted_iota(jnp.int32, (BN, NG), 1),
        1.0, 0.0)
    dn = (((0,), (0,)), ((), ()))
    acc[...] += lax.dot_general(onehot, h, dn,
                                preferred_element_type=jnp.float32,
                                precision=lax.Precision.HIGHEST)
    cnt[...] += lax.dot_general(onehot, jnp.ones((BN, D), jnp.float32), dn,
                                preferred_element_type=jnp.float32,
                                precision=lax.Precision.HIGHEST)

    @pl.when(i == pl.num_programs(0) - 1)
    def _():
        pooled = acc[...] / jnp.maximum(cnt[...], 1.0)
        logits = _kan(pooled, wcat_ref[...])
        m = jnp.max(logits, axis=1, keepdims=True)
        lse = jnp.log(jnp.sum(jnp.exp(logits - m), axis=1, keepdims=True))
        out_ref[...] = logits - m - lse


def _row_spec(width):
    return pl.BlockSpec((BN, width), lambda i: (i, 0))


def _full_spec(shape):
    return pl.BlockSpec(shape, lambda i: (0,) * len(shape))


def _tc_kan1(x, d0, d1, wcat):
    return pl.pallas_call(
        _kan1_body,
        grid=(GRID,),
        in_specs=[_row_spec(D), _row_spec(1), _row_spec(1),
                  _full_spec((8 * D, D))],
        out_specs=[_row_spec(D), _row_spec(1)],
        out_shape=[jax.ShapeDtypeStruct((N, D), jnp.float32),
                   jax.ShapeDtypeStruct((N, 1), jnp.float32)],
    )(x, d0, d1, wcat)


def _tc_mid(s0, s1, g, dinv, bias, wcat):
    return pl.pallas_call(
        _mid_body,
        grid=(GRID,),
        in_specs=[_row_spec(D), _row_spec(D), _row_spec(D), _row_spec(1),
                  _full_spec((1, D)), _full_spec((8 * D, D))],
        out_specs=_row_spec(D),
        out_shape=jax.ShapeDtypeStruct((N, D), jnp.float32),
    )(s0, s1, g, dinv, bias, wcat)


def _tc_final(s0, s1, g, dinv, bias, batch, wcat):
    return pl.pallas_call(
        _final_body,
        grid=(GRID,),
        in_specs=[_row_spec(D), _row_spec(D), _row_spec(D), _row_spec(1),
                  _full_spec((1, D)), _row_spec(1), _full_spec((8 * D, C))],
        out_specs=_full_spec((NG, C)),
        out_shape=jax.ShapeDtypeStruct((NG, C), jnp.float32),
        scratch_shapes=[pltpu.VMEM((NG, D), jnp.float32),
                        pltpu.VMEM((NG, D), jnp.float32)],
    )(s0, s1, g, dinv, bias, batch, wcat)


def _prep_w(base_w, spline_w, scaler):
    sw = spline_w * scaler[..., None]
    parts = [base_w.T] + [sw[:, :, j].T for j in range(NB)]
    return jnp.concatenate(parts, axis=0)


def kernel(x, base_w1, spline_w1, scaler1, bias1, base_w2, spline_w2,
           scaler2, bias2, base_w3, spline_w3, scaler3, edge_index, batch):
    row = edge_index[0].astype(jnp.int32).reshape(NW, CHS, KS)
    col = edge_index[1].astype(jnp.int32).reshape(NW, CHS, KS)
    wcat1 = _prep_w(base_w1, spline_w1, scaler1)
    wcat2 = _prep_w(base_w2, spline_w2, scaler2)
    wcat3 = _prep_w(base_w3, spline_w3, scaler3)

    zeros_row = jnp.zeros((RPTA, D), jnp.float32)

    sc_scatter = _sc_kernels()
    degp = sc_scatter(jnp.ones((N, D), jnp.float32), row, col, zeros_row)
    d0 = degp[0, :, 0:1]
    d1 = degp[1, :, 0:1]

    g1, dinv = _tc_kan1(x, d0, d1, wcat1)
    s1 = sc_scatter(g1, row, col, zeros_row)
    g2 = _tc_mid(s1[0], s1[1], g1, dinv, bias1.reshape(1, D), wcat2)
    s2 = sc_scatter(g2, row, col, zeros_row)
    return _tc_final(s2[0], s2[1], g2, dinv, bias2.reshape(1, D),
                     batch.astype(jnp.int32).reshape(N, 1), wcat3)
